# baseline (speedup 1.0000x reference)
import functools
import math

import jax
import jax.numpy as jnp
from jax import lax
from jax.experimental import pallas as pl
from jax.experimental.pallas import tpu as pltpu
from jax.experimental.pallas import tpu_sc as plsc


def _pos_encoding(seq_len, dim):
    position = jnp.arange(0, seq_len, dtype=jnp.float32)[:, None]
    div_term = jnp.exp(
        jnp.arange(0, dim, 2, dtype=jnp.float32) * -(math.log(10000.0) / dim)
    )
    pe = jnp.zeros((seq_len, dim), dtype=jnp.float32)
    pe = pe.at[:, 0::2].set(jnp.sin(position * div_term))
    pe = pe.at[:, 1::2].set(jnp.cos(position * div_term))
    return pe


@functools.partial(jax.jit, static_argnums=(3, 4))
def _sc_embed(idx, pe, table, batch, seq):
    dim = table.shape[1]
    NC, NS = 2, 16
    NW = NC * NS
    L = 16
    BT = 2
    blk = BT * 128
    n_bt = batch // 128
    n_blocks = seq * (n_bt // BT)
    per_w = n_blocks // NW
    n_dt = dim // 8

    mesh = plsc.VectorSubcoreMesh(core_axis_name="c", subcore_axis_name="s")

    @functools.partial(
        pl.kernel,
        mesh=mesh,
        out_type=jax.ShapeDtypeStruct((seq, n_dt, n_bt, 8, 128), jnp.float32),
        scratch_types=[
            pltpu.VMEM((seq, dim), jnp.float32),
            pltpu.VMEM((blk,), jnp.int32),
            pltpu.VMEM((blk,), jnp.int32),
            pltpu.VMEM((blk, dim), jnp.float32),
            pltpu.VMEM((blk, dim), jnp.float32),
            pltpu.VMEM((n_dt, BT, 8, 128), jnp.float32),
            pltpu.VMEM((n_dt, BT, 8, 128), jnp.float32),
            pltpu.SemaphoreType.DMA,
            pltpu.SemaphoreType.DMA,
            pltpu.SemaphoreType.DMA,
            pltpu.SemaphoreType.DMA,
            pltpu.SemaphoreType.DMA,
            pltpu.SemaphoreType.DMA,
        ],
        compiler_params=pltpu.CompilerParams(
            use_tc_tiling_on_sc=False, needs_layout_passes=False),
    )
    def body(idx_hbm, pe_hbm, table_hbm, out_hbm,
             pe_v, idx0, idx1, rows0, rows1, tiles0, tiles1,
             gs0, gs1, is0, is1, os0, os1):
        wid = lax.axis_index("s") * NC + lax.axis_index("c")
        first = wid * per_w
        pltpu.sync_copy(pe_hbm, pe_v)

        def issue_gather(idxb, rowsb, gsem):
            pltpu.async_copy(
                table_hbm.at[idxb.at[pl.ds(0, 128)]],
                rowsb.at[pl.ds(0, 128)], gsem)
            pltpu.async_copy(
                table_hbm.at[idxb.at[pl.ds(128, 128)]],
                rowsb.at[pl.ds(128, 128)], gsem)

        def idx_start(t):
            bid = first + t
            s = bid // (n_bt // BT)
            btp = bid % (n_bt // BT)
            return s, btp, s * batch + btp * blk

        bufs = ((idx0, rows0, tiles0, gs0, is0, os0),
                (idx1, rows1, tiles1, gs1, is1, os1))

        for b in range(2):
            idxb, rowsb, _, gsem, _, _ = bufs[b]
            _, _, start = idx_start(b)
            pltpu.sync_copy(idx_hbm.at[pl.ds(start, blk)], idxb)
            issue_gather(idxb, rowsb, gsem)

        iota = lax.iota(jnp.int32, L)

        def process(t, buf):
            idxb, rowsb, tilesb, gsem, isem, osem = buf
            s, btp, _ = idx_start(t)
            pltpu.make_async_copy(table_hbm.at[idxb], rowsb, gsem).wait()
            @pl.when(t + 2 < per_w)
            def _():
                _, _, start2 = idx_start(t + 2)
                pltpu.async_copy(idx_hbm.at[pl.ds(start2, blk)], idxb, isem)
            @pl.when(t >= 2)
            def _():
                for dt in range(n_dt):
                    pltpu.make_async_copy(
                        tilesb.at[dt],
                        out_hbm.at[0, dt, pl.ds(0, BT)], osem).wait()

            @plsc.parallel_loop(0, dim, 1, unroll=2)
            def _(d):
                dt = d // 8
                di = lax.rem(d, 8)
                pev = plsc.load_gather(
                    pe_v, (jnp.full((L,), s, jnp.int32),
                           jnp.full((L,), d, jnp.int32)))
                col = jnp.full((L,), d, jnp.int32)
                for g in range(blk // L):
                    rowsel = iota + g * L
                    v = plsc.load_gather(rowsb, (rowsel, col))
                    tilesb[dt, g // 8, di, pl.ds((g % 8) * L, L)] = v + pev

            for dt in range(n_dt):
                pltpu.async_copy(
                    tilesb.at[dt],
                    out_hbm.at[s, dt, pl.ds(btp * BT, BT)], osem)
            @pl.when(t + 2 < per_w)
            def _():
                pltpu.make_async_copy(
                    idx_hbm.at[pl.ds(0, blk)], idxb, isem).wait()
                issue_gather(idxb, rowsb, gsem)

        def step(g, carry):
            process(2 * g, bufs[0])
            process(2 * g + 1, bufs[1])
            return carry

        lax.fori_loop(0, per_w // 2, step, 0)

        for b in range(2):
            _, _, tilesb, _, _, osem = bufs[b]
            for dt in range(n_dt):
                pltpu.make_async_copy(
                    tilesb.at[dt],
                    out_hbm.at[0, dt, pl.ds(0, BT)], osem).wait()

    return body(idx, pe, table)


def kernel(x, table):
    batch, seq = x.shape
    dim = table.shape[1]
    idx = x.T.reshape(-1).astype(jnp.int32)
    pe = _pos_encoding(seq, dim)
    out5 = _sc_embed(idx, pe, table, batch, seq)
    return out5.transpose(2, 4, 0, 1, 3).reshape(batch, seq, dim)
